# baseline (speedup 1.0000x reference)
import functools

import jax
import jax.numpy as jnp
from jax import lax
from jax.experimental import pallas as pl
from jax.experimental.pallas import tpu as pltpu
from jax.experimental.pallas import tpu_sc as plsc

_N = 10000
_E = 320000
_D = 128

_NC = 2
_NS = 16
_NW = _NC * _NS
_L = 16

_CK = 128
_NCHUNK = _E // _CK
_MAXSPAN = ((_NCHUNK + _NW - 1) // _NW) * _CK


def _dense_body(xa_ref, xb_ref, xc_ref, xd_ref, w1_ref, b1_ref, w2_ref, s_ref):
    for x_ref, lo in ((xa_ref, 0), (xb_ref, _HB), (xc_ref, 2 * _HB),
                      (xd_ref, 3 * _HB)):
        h = jnp.tanh(
            lax.dot_general(
                x_ref[...], w1_ref[...], (((1,), (0,)), ((), ())),
                preferred_element_type=jnp.float32,
            )
            + b1_ref[...]
        )
        s_ref[:, lo:lo + _HB] = lax.dot_general(
            w2_ref[...], h, (((1,), (1,)), ((), ())),
            preferred_element_type=jnp.float32,
        )


_NP = 10240
_HB = 1280


def _dense_call(x, W1, b1_2d, w2_row):
    blk = 4 * _HB
    return pl.pallas_call(
        _dense_body,
        grid=(_NP // blk,),
        in_specs=[
            pl.BlockSpec((_HB, _D), lambda i: (4 * i, 0)),
            pl.BlockSpec((_HB, _D), lambda i: (4 * i + 1, 0)),
            pl.BlockSpec((_HB, _D), lambda i: (4 * i + 2, 0)),
            pl.BlockSpec((_HB, _D), lambda i: (4 * i + 3, 0)),
            pl.BlockSpec((_D, _D), lambda i: (0, 0)),
            pl.BlockSpec((1, _D), lambda i: (0, 0)),
            pl.BlockSpec((1, _D), lambda i: (0, 0)),
        ],
        out_specs=pl.BlockSpec((1, blk), lambda i: (0, i)),
        out_shape=jax.ShapeDtypeStruct((1, _NP), jnp.float32),
    )(x, x, x, x, W1, b1_2d, w2_row)


_sc_mesh = plsc.VectorSubcoreMesh(core_axis_name="c", subcore_axis_name="s")


@functools.partial(
    pl.kernel,
    out_type=jax.ShapeDtypeStruct((_NW, _N), jnp.float32),
    mesh=_sc_mesh,
    scratch_types=[
        pltpu.VMEM((_NP,), jnp.float32),
        pltpu.VMEM((2, _MAXSPAN), jnp.int32),
        pltpu.VMEM((_MAXSPAN,), jnp.float32),
        pltpu.VMEM((_N,), jnp.float32),
        pltpu.VMEM((_N,), jnp.float32),
        pltpu.SemaphoreType.DMA,
        pltpu.SemaphoreType.DMA,
        pltpu.SemaphoreType.DMA,
    ],
    compiler_params=pltpu.CompilerParams(needs_layout_passes=False),
)
def _sparse_kernel(s_hbm, ei_hbm, val_hbm, out_hbm,
                   s_v, ei_v, val_v, acc_v, acc2_v, sem_s, se0, sv0):
    cid = lax.axis_index("c")
    sid = lax.axis_index("s")
    wid = cid * _NS + sid
    start = (625 * wid) >> 3
    end = (625 * (wid + 1)) >> 3
    n16 = (end - start) * (_CK // _L)
    base = start * _CK

    cp0 = pltpu.async_copy(s_hbm.at[0], s_v, sem_s)
    cp1 = pltpu.async_copy(ei_hbm.at[:, pl.ds(base, _MAXSPAN)], ei_v, se0)
    cp2 = pltpu.async_copy(val_hbm.at[pl.ds(base, _MAXSPAN)], val_v, sv0)

    @plsc.parallel_loop(0, _N // _L, unroll=5)
    def _zero(i):
        acc_v[pl.ds(i * _L, _L)] = jnp.zeros((_L,), jnp.float32)
        acc2_v[pl.ds(i * _L, _L)] = jnp.zeros((_L,), jnp.float32)

    cp0.wait()
    cp1.wait()
    cp2.wait()

    @plsc.parallel_loop(0, n16 >> 1, unroll=8)
    def _edge(i):
        for acc, off in ((acc_v, 2 * i * _L), (acc2_v, (2 * i + 1) * _L)):
            r = ei_v[0, pl.ds(off, _L)]
            c = ei_v[1, pl.ds(off, _L)]
            v = val_v[pl.ds(off, _L)]
            g = plsc.load_gather(s_v, [c])
            plsc.addupdate_scatter(acc, [r], g * v)

    @plsc.parallel_loop(0, _N // _L, unroll=5)
    def _merge(i):
        off = i * _L
        acc_v[pl.ds(off, _L)] = acc_v[pl.ds(off, _L)] + acc2_v[pl.ds(off, _L)]

    pltpu.sync_copy(acc_v, out_hbm.at[wid])


def _reduce_body(p_ref, b2_ref, o_ref):
    o_ref[...] = jnp.sum(p_ref[...], axis=0, keepdims=True) + b2_ref[...]


def _reduce_call(partials, b2_2d):
    return pl.pallas_call(
        _reduce_body,
        in_specs=[
            pl.BlockSpec((_NW, _N), lambda: (0, 0)),
            pl.BlockSpec((1, 1), lambda: (0, 0)),
        ],
        out_specs=pl.BlockSpec((1, _N), lambda: (0, 0)),
        out_shape=jax.ShapeDtypeStruct((1, _N), jnp.float32),
    )(partials, b2_2d)


def kernel(x, adj_edge_index, adj_values, W1, b1, W2, b2):
    s = _dense_call(x, W1, b1.reshape(1, _D), W2.reshape(1, _D))
    partials = _sparse_kernel(s, adj_edge_index, adj_values)
    out = _reduce_call(partials, b2.reshape(1, 1))
    return out.reshape(_N, 1)


# speedup vs baseline: 1.0296x; 1.0296x over previous
import functools

import jax
import jax.numpy as jnp
from jax import lax
from jax.experimental import pallas as pl
from jax.experimental.pallas import tpu as pltpu
from jax.experimental.pallas import tpu_sc as plsc

_N = 10000
_E = 320000
_D = 128

_NC = 2
_NS = 16
_NW = _NC * _NS
_L = 16

_CK = 128
_NCHUNK = _E // _CK
_MAXSPAN = ((_NCHUNK + _NW - 1) // _NW) * _CK


def _dense_body(xa_ref, xb_ref, w1_ref, b1_ref, w2_ref, s_ref):
    for x_ref, lo in ((xa_ref, 0), (xb_ref, _HB)):
        h = jnp.tanh(
            lax.dot_general(
                x_ref[...], w1_ref[...], (((1,), (0,)), ((), ())),
                preferred_element_type=jnp.float32,
            )
            + b1_ref[...]
        )
        s_ref[:, lo:lo + _HB] = lax.dot_general(
            w2_ref[...], h, (((1,), (1,)), ((), ())),
            preferred_element_type=jnp.float32,
        )


_NP = 10240
_HB = 2560


def _dense_call(x, W1, b1_2d, w2_row):
    blk = 2 * _HB
    return pl.pallas_call(
        _dense_body,
        grid=(_NP // blk,),
        in_specs=[
            pl.BlockSpec((_HB, _D), lambda i: (2 * i, 0)),
            pl.BlockSpec((_HB, _D), lambda i: (2 * i + 1, 0)),
            pl.BlockSpec((_D, _D), lambda i: (0, 0)),
            pl.BlockSpec((1, _D), lambda i: (0, 0)),
            pl.BlockSpec((1, _D), lambda i: (0, 0)),
        ],
        out_specs=pl.BlockSpec((1, blk), lambda i: (0, i)),
        out_shape=jax.ShapeDtypeStruct((1, _NP), jnp.float32),
    )(x, x, W1, b1_2d, w2_row)


_sc_mesh = plsc.VectorSubcoreMesh(core_axis_name="c", subcore_axis_name="s")


@functools.partial(
    pl.kernel,
    out_type=jax.ShapeDtypeStruct((_NW, _N), jnp.float32),
    mesh=_sc_mesh,
    scratch_types=[
        pltpu.VMEM((_NP,), jnp.float32),
        pltpu.VMEM((2, _MAXSPAN), jnp.int32),
        pltpu.VMEM((_MAXSPAN,), jnp.float32),
        pltpu.VMEM((_N,), jnp.float32),
        pltpu.SemaphoreType.DMA,
        pltpu.SemaphoreType.DMA,
        pltpu.SemaphoreType.DMA,
    ],
    compiler_params=pltpu.CompilerParams(needs_layout_passes=False),
)
def _sparse_kernel(s_hbm, ei_hbm, val_hbm, out_hbm,
                   s_v, ei_v, val_v, acc_v, sem_s, se0, sv0):
    cid = lax.axis_index("c")
    sid = lax.axis_index("s")
    wid = cid * _NS + sid
    start = (625 * wid) >> 3
    end = (625 * (wid + 1)) >> 3
    n16 = (end - start) * (_CK // _L)
    base = start * _CK

    cp0 = pltpu.async_copy(s_hbm.at[0], s_v, sem_s)
    cp1 = pltpu.async_copy(ei_hbm.at[:, pl.ds(base, _MAXSPAN)], ei_v, se0)
    cp2 = pltpu.async_copy(val_hbm.at[pl.ds(base, _MAXSPAN)], val_v, sv0)

    @plsc.parallel_loop(0, _N // _L, unroll=5)
    def _zero(i):
        acc_v[pl.ds(i * _L, _L)] = jnp.zeros((_L,), jnp.float32)

    cp0.wait()
    cp1.wait()
    cp2.wait()

    @plsc.parallel_loop(0, n16, unroll=16)
    def _edge(i):
        off = i * _L
        r = ei_v[0, pl.ds(off, _L)]
        c = ei_v[1, pl.ds(off, _L)]
        v = val_v[pl.ds(off, _L)]
        g = plsc.load_gather(s_v, [c])
        plsc.addupdate_scatter(acc_v, [r], g * v)

    pltpu.sync_copy(acc_v, out_hbm.at[wid])


def _reduce_body(p_ref, b2_ref, o_ref):
    o_ref[...] = jnp.sum(p_ref[...], axis=0, keepdims=True) + b2_ref[...]


def _reduce_call(partials, b2_2d):
    return pl.pallas_call(
        _reduce_body,
        in_specs=[
            pl.BlockSpec((_NW, _N), lambda: (0, 0)),
            pl.BlockSpec((1, 1), lambda: (0, 0)),
        ],
        out_specs=pl.BlockSpec((1, _N), lambda: (0, 0)),
        out_shape=jax.ShapeDtypeStruct((1, _N), jnp.float32),
    )(partials, b2_2d)


def kernel(x, adj_edge_index, adj_values, W1, b1, W2, b2):
    s = _dense_call(x, W1, b1.reshape(1, _D), W2.reshape(1, _D))
    partials = _sparse_kernel(s, adj_edge_index, adj_values)
    out = _reduce_call(partials, b2.reshape(1, 1))
    return out.reshape(_N, 1)
